# baseline (speedup 1.0000x reference)
import jax
import jax.numpy as jnp
import numpy as np
from jax import lax
from jax.experimental import pallas as pl
from jax.experimental.pallas import tpu as pltpu
from jax.experimental.pallas import tpu_sc as plsc

D = 64
SCALE = 8.0
NC = 2
NS = 16
NW = NC * NS
SEQ = 50
BATCH = 16384
B_TOTAL = BATCH * SEQ
PER_W = B_TOTAL // NW
CHUNK = 512
NCHUNK = PER_W // CHUNK
LANES = 16
WBLK = BATCH // 128
SLABS = CHUNK // 128

def _transpose_scale(rows, out_t, b):
    iota = lax.iota(jnp.int32, LANES)
    for s in range(SLABS):
        @plsc.parallel_loop(0, D, unroll=2)
        def c_body(c):
            cb = c // 8
            ci = c % 8
            cv = jnp.full((LANES,), c, jnp.int32)
            for ig in range(128 // LANES):
                rv = iota + (128 * s + LANES * ig)
                v = plsc.load_gather(rows[b], [rv, cv])
                out_t[cb, s, ci, pl.ds(LANES * ig, LANES)] = v * SCALE


def _body(x_hbm, lut_hbm, out_hbm, idx_all, rows0, rows1, out_t, g0, g1):
    wid = lax.axis_index("s") * NC + lax.axis_index("c")
    base = wid * PER_W

    pltpu.sync_copy(x_hbm.at[pl.ds(base, PER_W)], idx_all)

    rows = (rows0, rows1)
    sems = (g0, g1)

    def start_gather(ci, b):
        pltpu.make_async_copy(
            lut_hbm.at[idx_all.at[pl.ds(ci * CHUNK, CHUNK)]],
            rows[b], sems[b]).start()

    def wait_gather(b):
        pltpu.make_async_copy(
            lut_hbm.at[idx_all.at[pl.ds(0, CHUNK)]],
            rows[b], sems[b]).wait()

    def process(ci, b):
        wait_gather(b)
        _transpose_scale(rows, out_t, b)
        b0 = base + ci * CHUNK
        j = b0 // BATCH
        w0 = (b0 % BATCH) // 128
        for cb in range(D // 8):
            pltpu.sync_copy(out_t.at[cb],
                            out_hbm.at[j, cb, pl.ds(w0, SLABS)])

    start_gather(0, 0)
    start_gather(1, 1)

    def pair_body(jj, _):
        for b in range(2):
            ci = 2 * jj + b
            process(ci, b)
            start_gather(ci + 2, b)
        return 0
    lax.fori_loop(0, NCHUNK // 2 - 1, pair_body, 0)

    process(NCHUNK - 2, 0)
    process(NCHUNK - 1, 1)


def _embed(x2_flat, lut2):
    mesh = plsc.VectorSubcoreMesh(core_axis_name="c", subcore_axis_name="s")
    return pl.kernel(
        _body,
        out_type=jax.ShapeDtypeStruct((SEQ, D // 8, WBLK, 8, 128),
                                      jnp.float32),
        mesh=mesh,
        compiler_params=pltpu.CompilerParams(
            use_tc_tiling_on_sc=False, needs_layout_passes=False),
        scratch_types=[
            pltpu.VMEM((PER_W,), jnp.int32),
            pltpu.VMEM((CHUNK, D), jnp.float32),
            pltpu.VMEM((CHUNK, D), jnp.float32),
            pltpu.VMEM((D // 8, SLABS, 8, 128), jnp.float32),
            pltpu.SemaphoreType.DMA,
            pltpu.SemaphoreType.DMA,
        ],
    )(x2_flat, lut2)


def kernel(x, lut):
    x2 = x.T.reshape(-1).astype(jnp.int32)
    out5 = _embed(x2, lut)
    return out5.transpose(2, 4, 0, 1, 3).reshape(BATCH, SEQ, D)
